# baseline (speedup 1.0000x reference)
import functools

import jax
import jax.numpy as jnp
import numpy as np
from jax import lax
from jax.experimental import pallas as pl
from jax.experimental.pallas import tpu as pltpu
from jax.experimental.pallas import tpu_sc as plsc

_N_TRAIN = 128
_K = 16
_L = 16
_W = 256
_QB_NORM = 4096
_QB_COMB = 8192
_HI_MASK = np.uint32(0xFFFFFF80)
_LO_MASK = np.uint32(127)


def _sort16(k, descending):
    return plsc.sort_key_val(k, k, descending=descending)[0]


def _merge16(ak, bk, direction):
    nk = jnp.minimum(ak, bk)
    if direction is not None:
        nk = _sort16(nk, direction)
    return nk


def _sc_select(ged, y, n_query):
    info = plsc.get_sparse_core_info()
    nc, ns = info.num_cores, info.num_subcores
    q_per_w = n_query // (nc * ns)
    n_tiles = q_per_w // _W
    n_leaves = _N_TRAIN // _L

    mesh = plsc.VectorSubcoreMesh(core_axis_name="c", subcore_axis_name="s")

    @functools.partial(
        pl.kernel,
        mesh=mesh,
        out_type=(
            jax.ShapeDtypeStruct((_K, n_query), jnp.float32),
            jax.ShapeDtypeStruct((_K, n_query), jnp.float32),
        ),
        scratch_types=[
            pltpu.VMEM((2, _N_TRAIN, _W), jnp.float32),
            pltpu.VMEM((_K, _W), jnp.float32),
            pltpu.VMEM((_K, _W), jnp.float32),
            pltpu.VMEM((_N_TRAIN,), jnp.float32),
            pltpu.SemaphoreType.DMA,
            pltpu.SemaphoreType.DMA,
        ],
        compiler_params=pltpu.CompilerParams(
            use_tc_tiling_on_sc=True, needs_layout_passes=False
        ),
    )
    def sc_knn(ged_hbm, y_hbm, kk_hbm, pp_hbm, tile_v, kv_v, pv_v, y_v,
               sem0, sem1):
        wid = lax.axis_index("s") * nc + lax.axis_index("c")
        pltpu.sync_copy(y_hbm, y_v)
        iota = lax.iota(jnp.int32, _L)
        row_idx = [iota + _L * j for j in range(n_leaves)]
        row_idx_u = [r.astype(jnp.uint32) for r in row_idx]
        q0 = wid * q_per_w
        sems = (sem0, sem1)

        def in_copy(t, slot):
            return pltpu.make_async_copy(
                ged_hbm.at[:, pl.ds(q0 + t * _W, _W)],
                tile_v.at[slot],
                sems[slot],
            )

        in_copy(0, 0).start()

        def do_tile(t, slot):
            in_copy(t, slot).wait()
            buf = tile_v.at[slot]

            @plsc.parallel_loop(0, _W, 1, unroll=6)
            def q_body(q):
                col = jnp.full((_L,), q, jnp.int32)
                vs = [
                    plsc.load_gather(buf, [row_idx[j], col])
                    for j in range(n_leaves)
                ]
                kv = [
                    _sort16(
                        (plsc.bitcast(vs[j], jnp.uint32) & _HI_MASK)
                        | row_idx_u[j],
                        bool(j & 1),
                    )
                    for j in range(n_leaves)
                ]
                m0 = _merge16(kv[0], kv[1], False)
                m1 = _merge16(kv[2], kv[3], True)
                m2 = _merge16(kv[4], kv[5], False)
                m3 = _merge16(kv[6], kv[7], True)
                p0 = _merge16(m0, m1, False)
                p1 = _merge16(m2, m3, True)
                fki = _merge16(p0, p1, None)

                fk = plsc.bitcast(fki, jnp.float32)
                fp = plsc.load_gather(
                    y_v, [(fki & _LO_MASK).astype(jnp.int32)]
                )
                plsc.store_scatter(kv_v, [iota, col], fk)
                plsc.store_scatter(pv_v, [iota, col], fp)

            pltpu.sync_copy(kv_v, kk_hbm.at[:, pl.ds(q0 + t * _W, _W)])
            pltpu.sync_copy(pv_v, pp_hbm.at[:, pl.ds(q0 + t * _W, _W)])

        def pair_body(g, carry):
            t = g * 2
            in_copy(t + 1, 1).start()
            do_tile(t, 0)

            @pl.when(t + 2 < n_tiles)
            def _():
                in_copy(t + 2, 0).start()

            do_tile(t + 1, 1)
            return carry

        lax.fori_loop(0, n_tiles // 2, pair_body, 0)

    return sc_knn(ged, y)


def _tc_rinv(ged, n_query):
    def body(g_ref, o_ref):
        x = g_ref[...]
        s = jnp.sum(x * x, axis=0, keepdims=True)
        o_ref[...] = 1.0 / jnp.maximum(jnp.sqrt(s), 1e-12)

    return pl.pallas_call(
        body,
        grid=(n_query // _QB_NORM,),
        in_specs=[
            pl.BlockSpec((_N_TRAIN, _QB_NORM), lambda i: (0, i)),
        ],
        out_specs=pl.BlockSpec((1, _QB_NORM), lambda i: (0, i)),
        out_shape=jax.ShapeDtypeStruct((1, n_query), jnp.float32),
    )(ged)


def _tc_combine(kk, pp, rinv, n_query):
    def body(k_ref, p_ref, r_ref, o_ref):
        sim = 1.0 / (k_ref[...] * r_ref[...] + 1.0)
        num = jnp.sum(sim * p_ref[...], axis=0, keepdims=True)
        den = jnp.sum(sim, axis=0, keepdims=True)
        o_ref[...] = num / den

    return pl.pallas_call(
        body,
        grid=(n_query // _QB_COMB,),
        in_specs=[
            pl.BlockSpec((_K, _QB_COMB), lambda i: (0, i)),
            pl.BlockSpec((_K, _QB_COMB), lambda i: (0, i)),
            pl.BlockSpec((1, _QB_COMB), lambda i: (0, i)),
        ],
        out_specs=pl.BlockSpec((1, _QB_COMB), lambda i: (0, i)),
        out_shape=jax.ShapeDtypeStruct((1, n_query), jnp.float32),
    )(kk, pp, rinv)


def kernel(ged, y):
    n_train, n_query = ged.shape
    kk, pp = _sc_select(ged, y, n_query)
    rinv = _tc_rinv(ged, n_query)
    out = _tc_combine(kk, pp, rinv, n_query)
    return out.reshape(n_query)


# speedup vs baseline: 1.0227x; 1.0227x over previous
import functools

import jax
import jax.numpy as jnp
import numpy as np
from jax import lax
from jax.experimental import pallas as pl
from jax.experimental.pallas import tpu as pltpu
from jax.experimental.pallas import tpu_sc as plsc

_N_TRAIN = 128
_K = 16
_L = 16
_W = 256
_QB_NORM = 4096
_QB_COMB = 8192
_HI_MASK = np.uint32(0xFFFFFF80)
_LO_MASK = np.uint32(127)


def _sort16(k, descending):
    return plsc.sort_key_val(k, k, descending=descending)[0]


def _merge16(ak, bk, direction):
    nk = jnp.minimum(ak, bk)
    if direction is not None:
        nk = _sort16(nk, direction)
    return nk


def _sc_select(ged, y, n_query):
    info = plsc.get_sparse_core_info()
    nc, ns = info.num_cores, info.num_subcores
    q_per_w = n_query // (nc * ns)
    n_tiles = q_per_w // _W
    n_leaves = _N_TRAIN // _L

    mesh = plsc.VectorSubcoreMesh(core_axis_name="c", subcore_axis_name="s")

    @functools.partial(
        pl.kernel,
        mesh=mesh,
        out_type=(
            jax.ShapeDtypeStruct((_K, n_query), jnp.float32),
            jax.ShapeDtypeStruct((_K, n_query), jnp.float32),
        ),
        scratch_types=[
            pltpu.VMEM((2, _N_TRAIN, _W), jnp.float32),
            pltpu.VMEM((_K, _W), jnp.float32),
            pltpu.VMEM((_K, _W), jnp.float32),
            pltpu.VMEM((_N_TRAIN,), jnp.float32),
            pltpu.SemaphoreType.DMA,
            pltpu.SemaphoreType.DMA,
        ],
        compiler_params=pltpu.CompilerParams(
            use_tc_tiling_on_sc=True, needs_layout_passes=False
        ),
    )
    def sc_knn(ged_hbm, y_hbm, kk_hbm, pp_hbm, tile_v, kv_v, pv_v, y_v,
               sem0, sem1):
        wid = lax.axis_index("s") * nc + lax.axis_index("c")
        pltpu.sync_copy(y_hbm, y_v)
        iota = lax.iota(jnp.int32, _L)
        row_idx = [iota + _L * j for j in range(n_leaves)]
        row_idx_u = [r.astype(jnp.uint32) for r in row_idx]
        q0 = wid * q_per_w
        sems = (sem0, sem1)

        def in_copy(t, slot):
            return pltpu.make_async_copy(
                ged_hbm.at[:, pl.ds(q0 + t * _W, _W)],
                tile_v.at[slot],
                sems[slot],
            )

        in_copy(0, 0).start()

        def do_tile(t, slot):
            in_copy(t, slot).wait()
            buf = tile_v.at[slot]

            @plsc.parallel_loop(0, _W, 1, unroll=3)
            def q_body(q):
                col = jnp.full((_L,), q, jnp.int32)
                vs = [
                    plsc.load_gather(buf, [row_idx[j], col])
                    for j in range(n_leaves)
                ]
                kv = [
                    _sort16(
                        (plsc.bitcast(vs[j], jnp.uint32) & _HI_MASK)
                        | row_idx_u[j],
                        bool(j & 1),
                    )
                    for j in range(n_leaves)
                ]
                m0 = _merge16(kv[0], kv[1], False)
                m1 = _merge16(kv[2], kv[3], True)
                m2 = _merge16(kv[4], kv[5], False)
                m3 = _merge16(kv[6], kv[7], True)
                p0 = _merge16(m0, m1, False)
                p1 = _merge16(m2, m3, True)
                fki = _merge16(p0, p1, None)

                fk = plsc.bitcast(fki, jnp.float32)
                fp = plsc.load_gather(
                    y_v, [(fki & _LO_MASK).astype(jnp.int32)]
                )
                plsc.store_scatter(kv_v, [iota, col], fk)
                plsc.store_scatter(pv_v, [iota, col], fp)

            pltpu.sync_copy(kv_v, kk_hbm.at[:, pl.ds(q0 + t * _W, _W)])
            pltpu.sync_copy(pv_v, pp_hbm.at[:, pl.ds(q0 + t * _W, _W)])

        def pair_body(g, carry):
            t = g * 2
            in_copy(t + 1, 1).start()
            do_tile(t, 0)

            @pl.when(t + 2 < n_tiles)
            def _():
                in_copy(t + 2, 0).start()

            do_tile(t + 1, 1)
            return carry

        lax.fori_loop(0, n_tiles // 2, pair_body, 0)

    return sc_knn(ged, y)


def _tc_rinv(ged, n_query):
    def body(g_ref, o_ref):
        x = g_ref[...]
        s = jnp.sum(x * x, axis=0, keepdims=True)
        o_ref[...] = 1.0 / jnp.maximum(jnp.sqrt(s), 1e-12)

    return pl.pallas_call(
        body,
        grid=(n_query // _QB_NORM,),
        in_specs=[
            pl.BlockSpec((_N_TRAIN, _QB_NORM), lambda i: (0, i)),
        ],
        out_specs=pl.BlockSpec((1, _QB_NORM), lambda i: (0, i)),
        out_shape=jax.ShapeDtypeStruct((1, n_query), jnp.float32),
    )(ged)


def _tc_combine(kk, pp, rinv, n_query):
    def body(k_ref, p_ref, r_ref, o_ref):
        sim = 1.0 / (k_ref[...] * r_ref[...] + 1.0)
        num = jnp.sum(sim * p_ref[...], axis=0, keepdims=True)
        den = jnp.sum(sim, axis=0, keepdims=True)
        o_ref[...] = num / den

    return pl.pallas_call(
        body,
        grid=(n_query // _QB_COMB,),
        in_specs=[
            pl.BlockSpec((_K, _QB_COMB), lambda i: (0, i)),
            pl.BlockSpec((_K, _QB_COMB), lambda i: (0, i)),
            pl.BlockSpec((1, _QB_COMB), lambda i: (0, i)),
        ],
        out_specs=pl.BlockSpec((1, _QB_COMB), lambda i: (0, i)),
        out_shape=jax.ShapeDtypeStruct((1, n_query), jnp.float32),
    )(kk, pp, rinv)


def kernel(ged, y):
    n_train, n_query = ged.shape
    kk, pp = _sc_select(ged, y, n_query)
    rinv = _tc_rinv(ged, n_query)
    out = _tc_combine(kk, pp, rinv, n_query)
    return out.reshape(n_query)


# speedup vs baseline: 1.0610x; 1.0374x over previous
import functools

import jax
import jax.numpy as jnp
import numpy as np
from jax import lax
from jax.experimental import pallas as pl
from jax.experimental.pallas import tpu as pltpu
from jax.experimental.pallas import tpu_sc as plsc

_N_TRAIN = 128
_K = 16
_L = 16
_W = 256
_QB_NORM = 4096
_QB_COMB = 8192
_HI_MASK = np.uint32(0xFFFFFF80)
_LO_MASK = np.uint32(127)


def _sort16(k, descending):
    return plsc.sort_key_val(k, k, descending=descending)[0]


def _merge16(ak, bk, direction):
    nk = jnp.minimum(ak, bk)
    if direction is not None:
        nk = _sort16(nk, direction)
    return nk


def _sc_select(ged, y, n_query):
    info = plsc.get_sparse_core_info()
    nc, ns = info.num_cores, info.num_subcores
    q_per_w = n_query // (nc * ns)
    n_tiles = q_per_w // _W
    n_leaves = _N_TRAIN // _L

    mesh = plsc.VectorSubcoreMesh(core_axis_name="c", subcore_axis_name="s")

    @functools.partial(
        pl.kernel,
        mesh=mesh,
        out_type=(
            jax.ShapeDtypeStruct((_K, n_query), jnp.float32),
            jax.ShapeDtypeStruct((_K, n_query), jnp.float32),
        ),
        scratch_types=[
            pltpu.VMEM((2, _N_TRAIN, _W), jnp.float32),
            pltpu.VMEM((_K, _W), jnp.float32),
            pltpu.VMEM((_K, _W), jnp.float32),
            pltpu.VMEM((_N_TRAIN,), jnp.float32),
            pltpu.SemaphoreType.DMA,
            pltpu.SemaphoreType.DMA,
        ],
        compiler_params=pltpu.CompilerParams(
            use_tc_tiling_on_sc=True, needs_layout_passes=False
        ),
    )
    def sc_knn(ged_hbm, y_hbm, kk_hbm, pp_hbm, tile_v, kv_v, pv_v, y_v,
               sem0, sem1):
        wid = lax.axis_index("s") * nc + lax.axis_index("c")
        pltpu.sync_copy(y_hbm, y_v)
        iota = lax.iota(jnp.int32, _L)
        row_idx = [iota + _L * j for j in range(n_leaves)]
        row_idx_u = [r.astype(jnp.uint32) for r in row_idx]
        q0 = wid * q_per_w
        sems = (sem0, sem1)

        def in_copy(t, slot):
            return pltpu.make_async_copy(
                ged_hbm.at[:, pl.ds(q0 + t * _W, _W)],
                tile_v.at[slot],
                sems[slot],
            )

        in_copy(0, 0).start()

        def do_tile(t, slot):
            in_copy(t, slot).wait()
            buf = tile_v.at[slot]

            @plsc.parallel_loop(0, _W, 1, unroll=4)
            def q_body(q):
                col = jnp.full((_L,), q, jnp.int32)
                vs = [
                    plsc.load_gather(buf, [row_idx[j], col])
                    for j in range(n_leaves)
                ]
                kv = [
                    _sort16(
                        (plsc.bitcast(vs[j], jnp.uint32) & _HI_MASK)
                        | row_idx_u[j],
                        bool(j & 1),
                    )
                    for j in range(n_leaves)
                ]
                m0 = _merge16(kv[0], kv[1], False)
                m1 = _merge16(kv[2], kv[3], True)
                m2 = _merge16(kv[4], kv[5], False)
                m3 = _merge16(kv[6], kv[7], True)
                p0 = _merge16(m0, m1, False)
                p1 = _merge16(m2, m3, True)
                fki = _merge16(p0, p1, None)

                fk = plsc.bitcast(fki, jnp.float32)
                fp = plsc.load_gather(
                    y_v, [(fki & _LO_MASK).astype(jnp.int32)]
                )
                plsc.store_scatter(kv_v, [iota, col], fk)
                plsc.store_scatter(pv_v, [iota, col], fp)

            pltpu.sync_copy(kv_v, kk_hbm.at[:, pl.ds(q0 + t * _W, _W)])
            pltpu.sync_copy(pv_v, pp_hbm.at[:, pl.ds(q0 + t * _W, _W)])

        def pair_body(g, carry):
            t = g * 2
            in_copy(t + 1, 1).start()
            do_tile(t, 0)

            @pl.when(t + 2 < n_tiles)
            def _():
                in_copy(t + 2, 0).start()

            do_tile(t + 1, 1)
            return carry

        lax.fori_loop(0, n_tiles // 2, pair_body, 0)

    return sc_knn(ged, y)


def _tc_rinv(ged, n_query):
    def body(g_ref, o_ref):
        x = g_ref[...]
        s = jnp.sum(x * x, axis=0, keepdims=True)
        o_ref[...] = 1.0 / jnp.maximum(jnp.sqrt(s), 1e-12)

    return pl.pallas_call(
        body,
        grid=(n_query // _QB_NORM,),
        in_specs=[
            pl.BlockSpec((_N_TRAIN, _QB_NORM), lambda i: (0, i)),
        ],
        out_specs=pl.BlockSpec((1, _QB_NORM), lambda i: (0, i)),
        out_shape=jax.ShapeDtypeStruct((1, n_query), jnp.float32),
    )(ged)


def _tc_combine(kk, pp, rinv, n_query):
    def body(k_ref, p_ref, r_ref, o_ref):
        sim = 1.0 / (k_ref[...] * r_ref[...] + 1.0)
        num = jnp.sum(sim * p_ref[...], axis=0, keepdims=True)
        den = jnp.sum(sim, axis=0, keepdims=True)
        o_ref[...] = num / den

    return pl.pallas_call(
        body,
        grid=(n_query // _QB_COMB,),
        in_specs=[
            pl.BlockSpec((_K, _QB_COMB), lambda i: (0, i)),
            pl.BlockSpec((_K, _QB_COMB), lambda i: (0, i)),
            pl.BlockSpec((1, _QB_COMB), lambda i: (0, i)),
        ],
        out_specs=pl.BlockSpec((1, _QB_COMB), lambda i: (0, i)),
        out_shape=jax.ShapeDtypeStruct((1, n_query), jnp.float32),
    )(kk, pp, rinv)


def kernel(ged, y):
    n_train, n_query = ged.shape
    kk, pp = _sc_select(ged, y, n_query)
    rinv = _tc_rinv(ged, n_query)
    out = _tc_combine(kk, pp, rinv, n_query)
    return out.reshape(n_query)
